# baseline (speedup 1.0000x reference)
import functools

import jax
import jax.numpy as jnp
from jax import lax
from jax.experimental import pallas as pl
from jax.experimental.pallas import tpu as pltpu
from jax.experimental.pallas import tpu_sc as plsc

_N = 50000
_E = 800000
_B = 256
_NP = 51200
_RPT = _NP // 16
_NW = 32
_CH = 196
_IB = 14
_NB = _CH // _IB
_EWP = _CH * 128
_BS = 512
_NG = _NP // _BS


def _sc_agg(u, src_p, dst_p, zrows):
    mesh = plsc.VectorSubcoreMesh(
        core_axis_name="c", subcore_axis_name="s", num_cores=2, num_subcores=16
    )

    @functools.partial(
        pl.kernel,
        mesh=mesh,
        compiler_params=pltpu.CompilerParams(use_tc_tiling_on_sc=False),
        out_type=jax.ShapeDtypeStruct((2, _NP, 32), jnp.float32),
        scratch_types=[
            pltpu.VMEM((2, _IB, 128), jnp.int32),
            pltpu.VMEM((2, _IB, 128), jnp.int32),
            pltpu.VMEM((3, 128, 32), jnp.float32),
            pltpu.VMEM_SHARED((_NP, 32), jnp.float32),
            pltpu.SemaphoreType.DMA,
            pltpu.SemaphoreType.DMA,
            pltpu.SemaphoreType.DMA,
            pltpu.SemaphoreType.DMA,
            pltpu.SemaphoreType.DMA,
        ],
    )
    def k(u_hbm, src_hbm, dst_hbm, z_hbm, out_hbm, sidx, didx, rows, agg,
          isem, gsa, gsb, ssa, ssb):
        c = lax.axis_index("c")
        s = lax.axis_index("s")
        w = s * 2 + c
        pltpu.async_copy(src_hbm.at[w, pl.ds(0, _IB)], sidx.at[0], isem)
        pltpu.async_copy(dst_hbm.at[w, pl.ds(0, _IB)], didx.at[0], isem)
        pltpu.sync_copy(z_hbm, agg.at[pl.ds(s * _RPT, _RPT)])
        plsc.subcore_barrier()

        def gather(jp, i, sem):
            pltpu.async_copy(u_hbm.at[sidx.at[jp, i]], rows.at[lax.rem(i, 3)],
                             sem)

        def wait_gather(sem):
            pltpu.make_async_copy(u_hbm.at[pl.ds(0, 128)], rows.at[0],
                                  sem).wait()

        def scatter(jp, i, sem):
            pltpu.async_copy(rows.at[lax.rem(i, 3)], agg.at[didx.at[jp, i]],
                             sem, add=True)

        def wait_scatter(sem):
            pltpu.make_async_copy(rows.at[0], agg.at[pl.ds(0, 128)],
                                  sem).wait()

        def outer(j, carry):
            jp = lax.rem(j, 2)
            pltpu.make_async_copy(src_hbm.at[w, pl.ds(0, _IB)], sidx.at[jp],
                                  isem).wait()
            pltpu.make_async_copy(dst_hbm.at[w, pl.ds(0, _IB)], didx.at[jp],
                                  isem).wait()

            @pl.when(j + 1 < _NB)
            def _():
                jn = lax.rem(j + 1, 2)
                off = (j + 1) * _IB
                pltpu.async_copy(src_hbm.at[w, pl.ds(off, _IB)], sidx.at[jn],
                                 isem)
                pltpu.async_copy(dst_hbm.at[w, pl.ds(off, _IB)], didx.at[jn],
                                 isem)

            gather(jp, 0, gsa)
            gather(jp, 1, gsb)

            def inner(t, c2):
                i0 = 2 * t
                i1 = 2 * t + 1
                wait_gather(gsa)
                scatter(jp, i0, ssa)

                @pl.when(t >= 1)
                def _():
                    wait_scatter(ssb)

                @pl.when(i0 + 2 < _IB)
                def _():
                    gather(jp, i0 + 2, gsa)

                wait_gather(gsb)
                scatter(jp, i1, ssb)
                wait_scatter(ssa)

                @pl.when(i1 + 2 < _IB)
                def _():
                    gather(jp, i1 + 2, gsb)

                return c2

            lax.fori_loop(0, _IB // 2, inner, 0)
            wait_scatter(ssb)
            return carry

        lax.fori_loop(0, _NB, outer, 0)
        plsc.subcore_barrier()
        pltpu.sync_copy(
            agg.at[pl.ds(s * _RPT, _RPT)], out_hbm.at[c, pl.ds(s * _RPT, _RPT)]
        )

    return k(u, src_p, dst_p, zrows)


def _mm_u0(xd_p, w1):
    def body(x_ref, w_ref, o_ref):
        o_ref[...] = jnp.dot(x_ref[...], w_ref[...],
                             preferred_element_type=jnp.float32)

    return pl.pallas_call(
        body,
        grid=(_NG,),
        in_specs=[pl.BlockSpec((_BS, 55), lambda i: (i, 0)),
                  pl.BlockSpec((55, 32), lambda i: (0, 0))],
        out_specs=pl.BlockSpec((_BS, 32), lambda i: (i, 0)),
        out_shape=jax.ShapeDtypeStruct((_NP, 32), jnp.float32),
    )(xd_p, w1)


def _mid(u, a0, a1, b1, w2, b2, g, bb, w1n):
    def body(u_ref, a0_ref, a1_ref, b1_ref, w2_ref, b2_ref, g_ref, bb_ref,
             w1n_ref, o_ref):
        i = pl.program_id(0)
        t = jnp.maximum(u_ref[...] + a0_ref[...] + a1_ref[...] + b1_ref[...],
                        0.0)
        z = jnp.maximum(jnp.dot(t, w2_ref[...],
                                preferred_element_type=jnp.float32)
                        + b2_ref[...], 0.0)
        z = z * g_ref[...] + bb_ref[...]
        un = jnp.dot(z, w1n_ref[...], preferred_element_type=jnp.float32)
        rows = i * _BS + lax.broadcasted_iota(jnp.int32, (_BS, 32), 0)
        o_ref[...] = jnp.where(rows < _N, un, 0.0)

    vec = lambda: pl.BlockSpec((1, 32), lambda i: (0, 0))
    return pl.pallas_call(
        body,
        grid=(_NG,),
        in_specs=[pl.BlockSpec((_BS, 32), lambda i: (i, 0)),
                  pl.BlockSpec((_BS, 32), lambda i: (i, 0)),
                  pl.BlockSpec((_BS, 32), lambda i: (i, 0)),
                  vec(),
                  pl.BlockSpec((32, 32), lambda i: (0, 0)),
                  vec(), vec(), vec(),
                  pl.BlockSpec((32, 32), lambda i: (0, 0))],
        out_specs=pl.BlockSpec((_BS, 32), lambda i: (i, 0)),
        out_shape=jax.ShapeDtypeStruct((_NP, 32), jnp.float32),
    )(u, a0, a1, b1, w2, b2, g, bb, w1n)


def _last_pool(u, a0, a1, b1, w2, b2, g, bb, batch3):
    def body(u_ref, a0_ref, a1_ref, b1_ref, w2_ref, b2_ref, g_ref, bb_ref,
             bt_ref, o_ref):
        i = pl.program_id(0)
        t = jnp.maximum(u_ref[...] + a0_ref[...] + a1_ref[...] + b1_ref[...],
                        0.0)
        z = jnp.maximum(jnp.dot(t, w2_ref[...],
                                preferred_element_type=jnp.float32)
                        + b2_ref[...], 0.0)
        z = z * g_ref[...] + bb_ref[...]
        ids = bt_ref[0]
        oh = (ids == lax.broadcasted_iota(jnp.int32, (_B, _BS), 0))
        part = jnp.dot(oh.astype(jnp.float32), z,
                       preferred_element_type=jnp.float32)

        @pl.when(i == 0)
        def _():
            o_ref[...] = part

        @pl.when(i > 0)
        def _():
            o_ref[...] += part

    vec = lambda: pl.BlockSpec((1, 32), lambda i: (0, 0))
    return pl.pallas_call(
        body,
        grid=(_NG,),
        in_specs=[pl.BlockSpec((_BS, 32), lambda i: (i, 0)),
                  pl.BlockSpec((_BS, 32), lambda i: (i, 0)),
                  pl.BlockSpec((_BS, 32), lambda i: (i, 0)),
                  vec(),
                  pl.BlockSpec((32, 32), lambda i: (0, 0)),
                  vec(), vec(), vec(),
                  pl.BlockSpec((1, 1, _BS), lambda i: (i, 0, 0))],
        out_specs=pl.BlockSpec((_B, 32), lambda i: (0, 0)),
        out_shape=jax.ShapeDtypeStruct((_B, 32), jnp.float32),
    )(u, a0, a1, b1, w2, b2, g, bb, batch3)


def _prot_a(xt, wr4):
    def body(xt_ref, w_ref, o_ref):
        c = pl.program_id(0)
        oh = (xt_ref[...] == c).astype(jnp.float32)
        o_ref[0] = jnp.dot(oh, w_ref[...], preferred_element_type=jnp.float32)

    return pl.pallas_call(
        body,
        grid=(26,),
        in_specs=[pl.BlockSpec((_B, 1000), lambda c: (0, 0)),
                  pl.BlockSpec((1000, 256), lambda c: (0, 0))],
        out_specs=pl.BlockSpec((1, _B, 256), lambda c: (c, 0, 0)),
        out_shape=jax.ShapeDtypeStruct((26, _B, 256), jnp.float32),
    )(xt, wr4)


def _prot_b(mbig, emb, fxt_w, cb, fxt_b):
    def body(m_ref, e_ref, fw_ref, cb_ref, fb_ref, o_ref):
        i = pl.program_id(0)
        e = e_ref[...]
        ekc = jnp.concatenate([e[:, k:k + 121] for k in range(8)], axis=0)
        conv = jnp.dot(m_ref[0], ekc,
                       preferred_element_type=jnp.float32) + cb_ref[0, 0, 0]
        part = jnp.dot(conv, fw_ref[0], preferred_element_type=jnp.float32)

        @pl.when(i == 0)
        def _():
            o_ref[...] = part + fb_ref[...]

        @pl.when(i > 0)
        def _():
            o_ref[...] += part

    return pl.pallas_call(
        body,
        grid=(32,),
        in_specs=[pl.BlockSpec((1, _B, 208), lambda o: (o, 0, 0)),
                  pl.BlockSpec((26, 128), lambda o: (0, 0)),
                  pl.BlockSpec((1, 121, 128), lambda o: (o, 0, 0)),
                  pl.BlockSpec((1, 1, 1), lambda o: (o, 0, 0)),
                  pl.BlockSpec((1, 128), lambda o: (0, 0))],
        out_specs=pl.BlockSpec((_B, 128), lambda o: (0, 0)),
        out_shape=jax.ShapeDtypeStruct((_B, 128), jnp.float32),
    )(mbig, emb, fxt_w, cb, fxt_b)


def _head(pooled, fxd_w, fxd_b, xt_o, w1, b1, w2, b2, w3p, b3):
    def body(p_ref, fw_ref, fb_ref, xt_ref, w1_ref, b1_ref, w2_ref, b2_ref,
             w3_ref, b3_ref, o_ref):
        xd_o = jnp.maximum(
            jnp.dot(p_ref[...], fw_ref[...],
                    preferred_element_type=jnp.float32) + fb_ref[...], 0.0)
        xj = jnp.concatenate([xd_o, xt_ref[...]], axis=1)
        h2 = jnp.maximum(jnp.dot(xj, w1_ref[...],
                                 preferred_element_type=jnp.float32)
                         + b1_ref[...], 0.0)
        h2 = jnp.maximum(jnp.dot(h2, w2_ref[...],
                                 preferred_element_type=jnp.float32)
                         + b2_ref[...], 0.0)
        o_ref[...] = jnp.dot(h2, w3_ref[...],
                             preferred_element_type=jnp.float32) + b3_ref[...]

    full = lambda shp: pl.BlockSpec(shp, lambda: tuple(0 for _ in shp))
    return pl.pallas_call(
        body,
        in_specs=[full((_B, 32)), full((32, 128)), full((1, 128)),
                  full((_B, 128)), full((256, 1024)), full((1, 1024)),
                  full((1024, 256)), full((1, 256)), full((256, 128)),
                  full((1, 128))],
        out_specs=full((_B, 128)),
        out_shape=jax.ShapeDtypeStruct((_B, 128), jnp.float32),
    )(pooled, fxd_w, fxd_b, xt_o, w1, b1, w2, b2, w3p, b3)


def kernel(xd, edge_index, batch, xt, y, params):
    f32 = jnp.float32
    src = edge_index[0].astype(jnp.int32)
    dst = edge_index[1].astype(jnp.int32)
    npad = _NW * _EWP - _E
    padv = _N + (jnp.arange(npad, dtype=jnp.int32) % 1024)
    src_p = jnp.concatenate([src, padv]).reshape(_NW, _CH, 128)
    dst_p = jnp.concatenate([dst, padv]).reshape(_NW, _CH, 128)
    xd_p = jnp.pad(xd.astype(f32), ((0, _NP - _N), (0, 0)))
    batch3 = jnp.pad(batch.astype(jnp.int32), (0, _NP - _N),
                     constant_values=300).reshape(_NG, 1, _BS)
    zrows = jnp.zeros((_RPT, 32), f32)

    gin = params["gin"]
    row = lambda v: v.reshape(1, -1).astype(f32)

    u = _mm_u0(xd_p, gin[0]["w1"].astype(f32))
    pooled = None
    for l in range(5):
        aggp = _sc_agg(u, src_p, dst_p, zrows)
        a0, a1 = aggp[0], aggp[1]
        p = gin[l]
        if l < 4:
            u = _mid(u, a0, a1, row(p["b1"]), p["w2"].astype(f32),
                     row(p["b2"]), row(p["bn_g"]), row(p["bn_b"]),
                     gin[l + 1]["w1"].astype(f32))
        else:
            pooled = _last_pool(u, a0, a1, row(p["b1"]), p["w2"].astype(f32),
                                row(p["b2"]), row(p["bn_g"]), row(p["bn_b"]),
                                batch3)

    xt_i = xt.astype(jnp.int32)
    wr4 = params["conv_w"].astype(f32).transpose(1, 0, 2).reshape(1000, 256)
    m3 = _prot_a(xt_i, wr4)
    mbig = (m3.reshape(26, _B, 32, 8).transpose(2, 1, 3, 0)
            .reshape(32, _B, 208))
    xt_o = _prot_b(mbig, params["emb"].astype(f32),
                   params["fc1_xt_w"].astype(f32).reshape(32, 121, 128),
                   params["conv_b"].astype(f32).reshape(32, 1, 1),
                   row(params["fc1_xt_b"]))

    w3p = jnp.pad(params["cls_w3"].astype(f32), ((0, 0), (0, 127)))
    b3p = jnp.pad(params["cls_b3"].astype(f32).reshape(1, 1),
                  ((0, 0), (0, 127)))
    head = _head(pooled, params["fc1_xd_w"].astype(f32),
                 row(params["fc1_xd_b"]), xt_o,
                 params["cls_w1"].astype(f32), row(params["cls_b1"]),
                 params["cls_w2"].astype(f32), row(params["cls_b2"]),
                 w3p, b3p)
    out = head[:, 0]
    return (out, y)


# speedup vs baseline: 1.9487x; 1.9487x over previous
import functools

import jax
import jax.numpy as jnp
from jax import lax
from jax.experimental import pallas as pl
from jax.experimental.pallas import tpu as pltpu
from jax.experimental.pallas import tpu_sc as plsc

_N = 50000
_E = 800000
_B = 256
_NP = 51200
_RPT = _NP // 16
_NW = 32
_CH = 196
_IB = 14
_NB = _CH // _IB
_EWP = _CH * 128
_BS = 512
_NG = _NP // _BS
_RP = _NP // 4
_BSP = 512
_NGP = _RP // _BSP


def _sc_agg(u, src_p, dst_p, zrows):
    mesh = plsc.VectorSubcoreMesh(
        core_axis_name="c", subcore_axis_name="s", num_cores=2, num_subcores=16
    )

    @functools.partial(
        pl.kernel,
        mesh=mesh,
        compiler_params=pltpu.CompilerParams(use_tc_tiling_on_sc=False),
        out_type=jax.ShapeDtypeStruct((2, _NP, 32), jnp.float32),
        scratch_types=[
            pltpu.VMEM((2, _IB, 128), jnp.int32),
            pltpu.VMEM((2, _IB, 128), jnp.int32),
            pltpu.VMEM((3, 128, 32), jnp.float32),
            pltpu.VMEM_SHARED((_NP, 32), jnp.float32),
            pltpu.SemaphoreType.DMA,
            pltpu.SemaphoreType.DMA,
            pltpu.SemaphoreType.DMA,
            pltpu.SemaphoreType.DMA,
            pltpu.SemaphoreType.DMA,
        ],
    )
    def k(u_hbm, src_hbm, dst_hbm, z_hbm, out_hbm, sidx, didx, rows, agg,
          isem, gsa, gsb, ssa, ssb):
        c = lax.axis_index("c")
        s = lax.axis_index("s")
        w = s * 2 + c
        pltpu.async_copy(src_hbm.at[w, pl.ds(0, _IB)], sidx.at[0], isem)
        pltpu.async_copy(dst_hbm.at[w, pl.ds(0, _IB)], didx.at[0], isem)
        pltpu.sync_copy(z_hbm, agg.at[pl.ds(s * _RPT, _RPT)])
        plsc.subcore_barrier()

        def gather(jp, i, sem):
            pltpu.async_copy(u_hbm.at[sidx.at[jp, i]], rows.at[lax.rem(i, 3)],
                             sem)

        def wait_gather(sem):
            pltpu.make_async_copy(u_hbm.at[pl.ds(0, 128)], rows.at[0],
                                  sem).wait()

        def scatter(jp, i, sem):
            pltpu.async_copy(rows.at[lax.rem(i, 3)], agg.at[didx.at[jp, i]],
                             sem, add=True)

        def wait_scatter(sem):
            pltpu.make_async_copy(rows.at[0], agg.at[pl.ds(0, 128)],
                                  sem).wait()

        def outer(j, carry):
            jp = lax.rem(j, 2)
            pltpu.make_async_copy(src_hbm.at[w, pl.ds(0, _IB)], sidx.at[jp],
                                  isem).wait()
            pltpu.make_async_copy(dst_hbm.at[w, pl.ds(0, _IB)], didx.at[jp],
                                  isem).wait()

            @pl.when(j + 1 < _NB)
            def _():
                jn = lax.rem(j + 1, 2)
                off = (j + 1) * _IB
                pltpu.async_copy(src_hbm.at[w, pl.ds(off, _IB)], sidx.at[jn],
                                 isem)
                pltpu.async_copy(dst_hbm.at[w, pl.ds(off, _IB)], didx.at[jn],
                                 isem)

            gather(jp, 0, gsa)
            gather(jp, 1, gsb)

            def inner(t, c2):
                i0 = 2 * t
                i1 = 2 * t + 1
                wait_gather(gsa)
                scatter(jp, i0, ssa)

                @pl.when(t >= 1)
                def _():
                    wait_scatter(ssb)

                @pl.when(i0 + 2 < _IB)
                def _():
                    gather(jp, i0 + 2, gsa)

                wait_gather(gsb)
                scatter(jp, i1, ssb)
                wait_scatter(ssa)

                @pl.when(i1 + 2 < _IB)
                def _():
                    gather(jp, i1 + 2, gsb)

                return c2

            lax.fori_loop(0, _IB // 2, inner, 0)
            wait_scatter(ssb)
            return carry

        lax.fori_loop(0, _NB, outer, 0)
        plsc.subcore_barrier()
        pltpu.sync_copy(
            agg.at[pl.ds(s * _RPT, _RPT)], out_hbm.at[c, pl.ds(s * _RPT, _RPT)]
        )

    return k(u, src_p, dst_p, zrows)


def _mm_u0(xd_perm, w1):
    def body(x_ref, w_ref, o_ref):
        w = w_ref[...]
        cols = [jnp.dot(x_ref[b * _BSP:(b + 1) * _BSP], w,
                        preferred_element_type=jnp.float32)
                for b in range(4)]
        o_ref[...] = jnp.concatenate(cols, axis=1)

    return pl.pallas_call(
        body,
        grid=(_NGP,),
        in_specs=[pl.BlockSpec((4 * _BSP, 55), lambda i: (i, 0)),
                  pl.BlockSpec((55, 32), lambda i: (0, 0))],
        out_specs=pl.BlockSpec((_BSP, 128), lambda i: (i, 0)),
        out_shape=jax.ShapeDtypeStruct((_RP, 128), jnp.float32),
    )(xd_perm, w1)


def _pk_spec():
    return pl.BlockSpec((_BSP, 128), lambda i: (i, 0))


def _gin_z(u_ref, a_refs, b1_ref, w2_ref, b2_ref, g_ref, bb_ref):
    t = u_ref[...]
    for a_ref in a_refs:
        t = t + a_ref[0]
    t = jnp.maximum(t + b1_ref[...], 0.0)
    z = jnp.maximum(jnp.dot(t, w2_ref[...],
                            preferred_element_type=jnp.float32)
                    + b2_ref[...], 0.0)
    return z * g_ref[...] + bb_ref[...]


def _mid(u, aggp, b1, w2bd, b2, g, bb, w1bd):
    def body(u_ref, a0_ref, a1_ref, b1_ref, w2_ref, b2_ref, g_ref, bb_ref,
             w1n_ref, o_ref):
        i = pl.program_id(0)
        z = _gin_z(u_ref, [a0_ref, a1_ref], b1_ref, w2_ref,
                   b2_ref, g_ref, bb_ref)
        un = jnp.dot(z, w1n_ref[...], preferred_element_type=jnp.float32)
        node = (4 * (i * _BSP
                     + lax.broadcasted_iota(jnp.int32, (_BSP, 128), 0))
                + lax.broadcasted_iota(jnp.int32, (_BSP, 128), 1) // 32)
        o_ref[...] = jnp.where(node < _N, un, 0.0)

    vec = lambda: pl.BlockSpec((1, 128), lambda i: (0, 0))
    mat = lambda: pl.BlockSpec((128, 128), lambda i: (0, 0))
    agg_spec = lambda c: pl.BlockSpec((1, _BSP, 128), lambda i: (c, i, 0))
    return pl.pallas_call(
        body,
        grid=(_NGP,),
        in_specs=[_pk_spec(), agg_spec(0), agg_spec(1),
                  vec(), mat(), vec(), vec(), vec(), mat()],
        out_specs=_pk_spec(),
        out_shape=jax.ShapeDtypeStruct((_RP, 128), jnp.float32),
    )(u, aggp, aggp, b1, w2bd, b2, g, bb, w1bd)


def _last_pool(u, aggp, b1, w2bd, b2, g, bb, batch_pk):
    def body(u_ref, a0_ref, a1_ref, b1_ref, w2_ref, b2_ref, g_ref, bb_ref,
             bt_ref, o_ref):
        i = pl.program_id(0)
        z = _gin_z(u_ref, [a0_ref, a1_ref], b1_ref, w2_ref,
                   b2_ref, g_ref, bb_ref)
        bt = bt_ref[0]
        part = jnp.zeros((_B, 32), jnp.float32)
        for a in range(4):
            ids = bt[a:a + 1, :]
            oh = (ids == lax.broadcasted_iota(jnp.int32, (_B, _BSP), 0))
            part = part + jnp.dot(oh.astype(jnp.float32),
                                  z[:, 32 * a:32 * a + 32],
                                  preferred_element_type=jnp.float32)

        @pl.when(i == 0)
        def _():
            o_ref[...] = part

        @pl.when(i > 0)
        def _():
            o_ref[...] += part

    vec = lambda: pl.BlockSpec((1, 128), lambda i: (0, 0))
    mat = lambda: pl.BlockSpec((128, 128), lambda i: (0, 0))
    agg_spec = lambda c: pl.BlockSpec((1, _BSP, 128), lambda i: (c, i, 0))
    return pl.pallas_call(
        body,
        grid=(_NGP,),
        in_specs=[_pk_spec(), agg_spec(0), agg_spec(1),
                  vec(), mat(), vec(), vec(), vec(),
                  pl.BlockSpec((1, 4, _BSP), lambda i: (i, 0, 0))],
        out_specs=pl.BlockSpec((_B, 32), lambda i: (0, 0)),
        out_shape=jax.ShapeDtypeStruct((_B, 32), jnp.float32),
    )(u, aggp, aggp, b1, w2bd, b2, g, bb, batch_pk)


def _prot_a(xt, wr4):
    def body(xt_ref, w_ref, o_ref):
        c = pl.program_id(0)
        oh = (xt_ref[...] == c).astype(jnp.float32)
        o_ref[0] = jnp.dot(oh, w_ref[...], preferred_element_type=jnp.float32)

    return pl.pallas_call(
        body,
        grid=(26,),
        in_specs=[pl.BlockSpec((_B, 1000), lambda c: (0, 0)),
                  pl.BlockSpec((1000, 256), lambda c: (0, 0))],
        out_specs=pl.BlockSpec((1, _B, 256), lambda c: (c, 0, 0)),
        out_shape=jax.ShapeDtypeStruct((26, _B, 256), jnp.float32),
    )(xt, wr4)


def _prot_b(mbig, emb, fxt_w, cb, fxt_b):
    def body(m_ref, e_ref, fw_ref, cb_ref, fb_ref, o_ref):
        i = pl.program_id(0)
        e = e_ref[...]
        ekc = jnp.concatenate([e[:, k:k + 121] for k in range(8)], axis=0)
        conv = jnp.dot(m_ref[0], ekc,
                       preferred_element_type=jnp.float32) + cb_ref[0, 0, 0]
        part = jnp.dot(conv, fw_ref[0], preferred_element_type=jnp.float32)

        @pl.when(i == 0)
        def _():
            o_ref[...] = part + fb_ref[...]

        @pl.when(i > 0)
        def _():
            o_ref[...] += part

    return pl.pallas_call(
        body,
        grid=(32,),
        in_specs=[pl.BlockSpec((1, _B, 208), lambda o: (o, 0, 0)),
                  pl.BlockSpec((26, 128), lambda o: (0, 0)),
                  pl.BlockSpec((1, 121, 128), lambda o: (o, 0, 0)),
                  pl.BlockSpec((1, 1, 1), lambda o: (o, 0, 0)),
                  pl.BlockSpec((1, 128), lambda o: (0, 0))],
        out_specs=pl.BlockSpec((_B, 128), lambda o: (0, 0)),
        out_shape=jax.ShapeDtypeStruct((_B, 128), jnp.float32),
    )(mbig, emb, fxt_w, cb, fxt_b)


def _head(pooled, fxd_w, fxd_b, xt_o, w1, b1, w2, b2, w3p, b3):
    def body(p_ref, fw_ref, fb_ref, xt_ref, w1_ref, b1_ref, w2_ref, b2_ref,
             w3_ref, b3_ref, o_ref):
        xd_o = jnp.maximum(
            jnp.dot(p_ref[...], fw_ref[...],
                    preferred_element_type=jnp.float32) + fb_ref[...], 0.0)
        xj = jnp.concatenate([xd_o, xt_ref[...]], axis=1)
        h2 = jnp.maximum(jnp.dot(xj, w1_ref[...],
                                 preferred_element_type=jnp.float32)
                         + b1_ref[...], 0.0)
        h2 = jnp.maximum(jnp.dot(h2, w2_ref[...],
                                 preferred_element_type=jnp.float32)
                         + b2_ref[...], 0.0)
        o_ref[...] = jnp.dot(h2, w3_ref[...],
                             preferred_element_type=jnp.float32) + b3_ref[...]

    full = lambda shp: pl.BlockSpec(shp, lambda: tuple(0 for _ in shp))
    return pl.pallas_call(
        body,
        in_specs=[full((_B, 32)), full((32, 128)), full((1, 128)),
                  full((_B, 128)), full((256, 1024)), full((1, 1024)),
                  full((1024, 256)), full((1, 256)), full((256, 128)),
                  full((1, 128))],
        out_specs=full((_B, 128)),
        out_shape=jax.ShapeDtypeStruct((_B, 128), jnp.float32),
    )(pooled, fxd_w, fxd_b, xt_o, w1, b1, w2, b2, w3p, b3)


def kernel(xd, edge_index, batch, xt, y, params):
    f32 = jnp.float32
    src = edge_index[0].astype(jnp.int32)
    dst = edge_index[1].astype(jnp.int32)
    npad = _NW * _EWP - _E
    padv = _N + (jnp.arange(npad, dtype=jnp.int32) % 1024)
    src_p = jnp.concatenate([src, padv]).reshape(_NW, _CH, 128)
    dst_p = jnp.concatenate([dst, padv]).reshape(_NW, _CH, 128)
    xd_p = jnp.pad(xd.astype(f32), ((0, _NP - _N), (0, 0)))
    xd_perm = (xd_p.reshape(_NGP, _BSP, 4, 55).transpose(0, 2, 1, 3)
               .reshape(_NP, 55))
    batch_pk = (jnp.pad(batch.astype(jnp.int32), (0, _NP - _N),
                        constant_values=300)
                .reshape(_RP, 4).transpose(1, 0)
                .reshape(4, _NGP, _BSP).transpose(1, 0, 2))
    zrows = jnp.zeros((_RPT, 32), f32)

    gin = params["gin"]
    row = lambda v: v.reshape(1, -1).astype(f32)
    eye4 = jnp.eye(4, dtype=f32)
    tile4 = lambda v: jnp.tile(v.astype(f32).reshape(1, -1), (1, 4))
    bd = lambda m: jnp.kron(eye4, m.astype(f32))

    u = _mm_u0(xd_perm, gin[0]["w1"].astype(f32))
    pooled = None
    for l in range(5):
        aggp = _sc_agg(u.reshape(_NP, 32), src_p, dst_p, zrows)
        aggp = aggp.reshape(2, _RP, 128)
        p = gin[l]
        if l < 4:
            u = _mid(u, aggp, tile4(p["b1"]), bd(p["w2"]),
                     tile4(p["b2"]), tile4(p["bn_g"]), tile4(p["bn_b"]),
                     bd(gin[l + 1]["w1"]))
        else:
            pooled = _last_pool(u, aggp, tile4(p["b1"]), bd(p["w2"]),
                                tile4(p["b2"]), tile4(p["bn_g"]),
                                tile4(p["bn_b"]), batch_pk)

    xt_i = xt.astype(jnp.int32)
    wr4 = params["conv_w"].astype(f32).transpose(1, 0, 2).reshape(1000, 256)
    m3 = _prot_a(xt_i, wr4)
    mbig = (m3.reshape(26, _B, 32, 8).transpose(2, 1, 3, 0)
            .reshape(32, _B, 208))
    xt_o = _prot_b(mbig, params["emb"].astype(f32),
                   params["fc1_xt_w"].astype(f32).reshape(32, 121, 128),
                   params["conv_b"].astype(f32).reshape(32, 1, 1),
                   row(params["fc1_xt_b"]))

    w3p = jnp.pad(params["cls_w3"].astype(f32), ((0, 0), (0, 127)))
    b3p = jnp.pad(params["cls_b3"].astype(f32).reshape(1, 1),
                  ((0, 0), (0, 127)))
    head = _head(pooled, params["fc1_xd_w"].astype(f32),
                 row(params["fc1_xd_b"]), xt_o,
                 params["cls_w1"].astype(f32), row(params["cls_b1"]),
                 params["cls_w2"].astype(f32), row(params["cls_b2"]),
                 w3p, b3p)
    out = head[:, 0]
    return (out, y)


# speedup vs baseline: 2.2388x; 1.1489x over previous
import functools

import jax
import jax.numpy as jnp
from jax import lax
from jax.experimental import pallas as pl
from jax.experimental.pallas import tpu as pltpu
from jax.experimental.pallas import tpu_sc as plsc

_N = 50000
_E = 800000
_B = 256
_NP = 51200
_RPT = _NP // 16
_NW = 32
_CH = 196
_IB = 14
_NB = _CH // _IB
_EWP = _CH * 128
_BS = 512
_NG = _NP // _BS
_RP = _NP // 4
_BSP = 512
_NGP = _RP // _BSP


def _sc_agg(u, src_p, dst_p):
    mesh = plsc.VectorSubcoreMesh(
        core_axis_name="c", subcore_axis_name="s", num_cores=2, num_subcores=16
    )

    @functools.partial(
        pl.kernel,
        mesh=mesh,
        compiler_params=pltpu.CompilerParams(use_tc_tiling_on_sc=False),
        out_type=jax.ShapeDtypeStruct((2, _NP, 32), jnp.float32),
        scratch_types=[
            pltpu.VMEM((2, _IB, 128), jnp.int32),
            pltpu.VMEM((2, _IB, 128), jnp.int32),
            pltpu.VMEM((4, 128, 32), jnp.float32),
            pltpu.VMEM_SHARED((_NP, 32), jnp.float32),
            pltpu.SemaphoreType.DMA,
            [pltpu.SemaphoreType.DMA] * 4,
            [pltpu.SemaphoreType.DMA] * 4,
        ],
    )
    def k(u_hbm, src_hbm, dst_hbm, out_hbm, sidx, didx, rows, agg,
          isem, gsem, ssem):
        c = lax.axis_index("c")
        s = lax.axis_index("s")
        w = s * 2 + c
        pltpu.async_copy(src_hbm.at[w, pl.ds(0, _IB)], sidx.at[0], isem)
        pltpu.async_copy(dst_hbm.at[w, pl.ds(0, _IB)], didx.at[0], isem)
        z16 = jnp.zeros((16,), jnp.float32)

        def memset(r, carry):
            rows[0, r, pl.ds(0, 16)] = z16
            rows[0, r, pl.ds(16, 16)] = z16
            return carry

        lax.fori_loop(0, 128, memset, 0)

        def zero(t, carry):
            pltpu.sync_copy(rows.at[0, pl.ds(0, 128)],
                            agg.at[pl.ds(s * _RPT + t * 128, 128)])
            return carry

        lax.fori_loop(0, _RPT // 128, zero, 0)
        plsc.subcore_barrier()

        def gather(jp, i2, b):
            pltpu.async_copy(u_hbm.at[sidx.at[jp, i2]],
                             rows.at[b], gsem[b])

        def wait_gather(b):
            pltpu.make_async_copy(u_hbm.at[pl.ds(0, 128)], rows.at[b],
                                  gsem[b]).wait()

        def scatter(jp, i2, b):
            pltpu.async_copy(rows.at[b], agg.at[didx.at[jp, i2]],
                             ssem[b], add=True)

        def wait_scatter(b):
            pltpu.make_async_copy(rows.at[b], agg.at[pl.ds(0, 128)],
                                  ssem[b]).wait()

        def outer(j, carry):
            jp = lax.rem(j, 2)
            pltpu.make_async_copy(src_hbm.at[w, pl.ds(0, _IB)], sidx.at[jp],
                                  isem).wait()
            pltpu.make_async_copy(dst_hbm.at[w, pl.ds(0, _IB)], didx.at[jp],
                                  isem).wait()

            @pl.when(j + 1 < _NB)
            def _():
                jn = lax.rem(j + 1, 2)
                off = (j + 1) * _IB
                pltpu.async_copy(src_hbm.at[w, pl.ds(off, _IB)], sidx.at[jn],
                                 isem)
                pltpu.async_copy(dst_hbm.at[w, pl.ds(off, _IB)], didx.at[jn],
                                 isem)

            for i2 in range(3):
                gather(jp, i2, i2)
            for i2 in range(_IB):
                b = i2 % 4
                wait_gather(b)
                scatter(jp, i2, b)
                if i2 >= 1:
                    wait_scatter((i2 - 1) % 4)
                if i2 + 3 < _IB:
                    gather(jp, i2 + 3, (i2 + 3) % 4)
            wait_scatter((_IB - 1) % 4)
            return carry

        lax.fori_loop(0, _NB, outer, 0)
        plsc.subcore_barrier()
        pltpu.sync_copy(
            agg.at[pl.ds(s * _RPT, _RPT)], out_hbm.at[c, pl.ds(s * _RPT, _RPT)]
        )

    return k(u, src_p, dst_p)


def _mm_u0(xd_perm, w1):
    def body(x_ref, w_ref, o_ref):
        w = w_ref[...]
        cols = [jnp.dot(x_ref[b * _BSP:(b + 1) * _BSP], w,
                        preferred_element_type=jnp.float32)
                for b in range(4)]
        o_ref[...] = jnp.concatenate(cols, axis=1)

    return pl.pallas_call(
        body,
        grid=(_NGP,),
        in_specs=[pl.BlockSpec((4 * _BSP, 55), lambda i: (i, 0)),
                  pl.BlockSpec((55, 32), lambda i: (0, 0))],
        out_specs=pl.BlockSpec((_BSP, 128), lambda i: (i, 0)),
        out_shape=jax.ShapeDtypeStruct((_RP, 128), jnp.float32),
    )(xd_perm, w1)


def _pk_spec():
    return pl.BlockSpec((_BSP, 128), lambda i: (i, 0))


def _gin_z(u_ref, a_refs, b1_ref, w2_ref, b2_ref, g_ref, bb_ref):
    t = u_ref[...]
    for a_ref in a_refs:
        t = t + a_ref[0]
    t = jnp.maximum(t + b1_ref[...], 0.0)
    z = jnp.maximum(jnp.dot(t, w2_ref[...],
                            preferred_element_type=jnp.float32)
                    + b2_ref[...], 0.0)
    return z * g_ref[...] + bb_ref[...]


def _mid(u, aggp, b1, w2bd, b2, g, bb, w1bd):
    def body(u_ref, a0_ref, a1_ref, b1_ref, w2_ref, b2_ref, g_ref, bb_ref,
             w1n_ref, o_ref):
        i = pl.program_id(0)
        z = _gin_z(u_ref, [a0_ref, a1_ref], b1_ref, w2_ref,
                   b2_ref, g_ref, bb_ref)
        un = jnp.dot(z, w1n_ref[...], preferred_element_type=jnp.float32)
        node = (4 * (i * _BSP
                     + lax.broadcasted_iota(jnp.int32, (_BSP, 128), 0))
                + lax.broadcasted_iota(jnp.int32, (_BSP, 128), 1) // 32)
        o_ref[...] = jnp.where(node < _N, un, 0.0)

    vec = lambda: pl.BlockSpec((1, 128), lambda i: (0, 0))
    mat = lambda: pl.BlockSpec((128, 128), lambda i: (0, 0))
    agg_spec = lambda c: pl.BlockSpec((1, _BSP, 128), lambda i: (c, i, 0))
    return pl.pallas_call(
        body,
        grid=(_NGP,),
        in_specs=[_pk_spec(), agg_spec(0), agg_spec(1),
                  vec(), mat(), vec(), vec(), vec(), mat()],
        out_specs=_pk_spec(),
        out_shape=jax.ShapeDtypeStruct((_RP, 128), jnp.float32),
    )(u, aggp, aggp, b1, w2bd, b2, g, bb, w1bd)


def _last_pool(u, aggp, b1, w2bd, b2, g, bb, batch_pk):
    def body(u_ref, a0_ref, a1_ref, b1_ref, w2_ref, b2_ref, g_ref, bb_ref,
             bt_ref, o_ref):
        i = pl.program_id(0)
        z = _gin_z(u_ref, [a0_ref, a1_ref], b1_ref, w2_ref,
                   b2_ref, g_ref, bb_ref)
        bt = bt_ref[0]
        part = jnp.zeros((_B, 32), jnp.float32)
        for a in range(4):
            ids = bt[a:a + 1, :]
            oh = (ids == lax.broadcasted_iota(jnp.int32, (_B, _BSP), 0))
            part = part + jnp.dot(oh.astype(jnp.float32),
                                  z[:, 32 * a:32 * a + 32],
                                  preferred_element_type=jnp.float32)

        @pl.when(i == 0)
        def _():
            o_ref[...] = part

        @pl.when(i > 0)
        def _():
            o_ref[...] += part

    vec = lambda: pl.BlockSpec((1, 128), lambda i: (0, 0))
    mat = lambda: pl.BlockSpec((128, 128), lambda i: (0, 0))
    agg_spec = lambda c: pl.BlockSpec((1, _BSP, 128), lambda i: (c, i, 0))
    return pl.pallas_call(
        body,
        grid=(_NGP,),
        in_specs=[_pk_spec(), agg_spec(0), agg_spec(1),
                  vec(), mat(), vec(), vec(), vec(),
                  pl.BlockSpec((1, 4, _BSP), lambda i: (i, 0, 0))],
        out_specs=pl.BlockSpec((_B, 32), lambda i: (0, 0)),
        out_shape=jax.ShapeDtypeStruct((_B, 32), jnp.float32),
    )(u, aggp, aggp, b1, w2bd, b2, g, bb, batch_pk)


def _prot_a(xt, wr4):
    def body(xt_ref, w_ref, o_ref):
        c = pl.program_id(0)
        oh = (xt_ref[...] == c).astype(jnp.float32)
        o_ref[0] = jnp.dot(oh, w_ref[...], preferred_element_type=jnp.float32)

    return pl.pallas_call(
        body,
        grid=(26,),
        in_specs=[pl.BlockSpec((_B, 1000), lambda c: (0, 0)),
                  pl.BlockSpec((1000, 256), lambda c: (0, 0))],
        out_specs=pl.BlockSpec((1, _B, 256), lambda c: (c, 0, 0)),
        out_shape=jax.ShapeDtypeStruct((26, _B, 256), jnp.float32),
    )(xt, wr4)


def _prot_b(mbig, emb, fxt_w, cb, fxt_b):
    def body(m_ref, e_ref, fw_ref, cb_ref, fb_ref, o_ref):
        i = pl.program_id(0)
        e = e_ref[...]
        ekc = jnp.concatenate([e[:, k:k + 121] for k in range(8)], axis=0)
        conv = jnp.dot(m_ref[0], ekc,
                       preferred_element_type=jnp.float32) + cb_ref[0, 0, 0]
        part = jnp.dot(conv, fw_ref[0], preferred_element_type=jnp.float32)

        @pl.when(i == 0)
        def _():
            o_ref[...] = part + fb_ref[...]

        @pl.when(i > 0)
        def _():
            o_ref[...] += part

    return pl.pallas_call(
        body,
        grid=(32,),
        in_specs=[pl.BlockSpec((1, _B, 208), lambda o: (o, 0, 0)),
                  pl.BlockSpec((26, 128), lambda o: (0, 0)),
                  pl.BlockSpec((1, 121, 128), lambda o: (o, 0, 0)),
                  pl.BlockSpec((1, 1, 1), lambda o: (o, 0, 0)),
                  pl.BlockSpec((1, 128), lambda o: (0, 0))],
        out_specs=pl.BlockSpec((_B, 128), lambda o: (0, 0)),
        out_shape=jax.ShapeDtypeStruct((_B, 128), jnp.float32),
    )(mbig, emb, fxt_w, cb, fxt_b)


def _head(pooled, fxd_w, fxd_b, xt_o, w1, b1, w2, b2, w3p, b3):
    def body(p_ref, fw_ref, fb_ref, xt_ref, w1_ref, b1_ref, w2_ref, b2_ref,
             w3_ref, b3_ref, o_ref):
        xd_o = jnp.maximum(
            jnp.dot(p_ref[...], fw_ref[...],
                    preferred_element_type=jnp.float32) + fb_ref[...], 0.0)
        xj = jnp.concatenate([xd_o, xt_ref[...]], axis=1)
        h2 = jnp.maximum(jnp.dot(xj, w1_ref[...],
                                 preferred_element_type=jnp.float32)
                         + b1_ref[...], 0.0)
        h2 = jnp.maximum(jnp.dot(h2, w2_ref[...],
                                 preferred_element_type=jnp.float32)
                         + b2_ref[...], 0.0)
        o_ref[...] = jnp.dot(h2, w3_ref[...],
                             preferred_element_type=jnp.float32) + b3_ref[...]

    full = lambda shp: pl.BlockSpec(shp, lambda: tuple(0 for _ in shp))
    return pl.pallas_call(
        body,
        in_specs=[full((_B, 32)), full((32, 128)), full((1, 128)),
                  full((_B, 128)), full((256, 1024)), full((1, 1024)),
                  full((1024, 256)), full((1, 256)), full((256, 128)),
                  full((1, 128))],
        out_specs=full((_B, 128)),
        out_shape=jax.ShapeDtypeStruct((_B, 128), jnp.float32),
    )(pooled, fxd_w, fxd_b, xt_o, w1, b1, w2, b2, w3p, b3)


def kernel(xd, edge_index, batch, xt, y, params):
    f32 = jnp.float32
    src = edge_index[0].astype(jnp.int32)
    dst = edge_index[1].astype(jnp.int32)
    npad = _NW * _EWP - _E
    padv = _N + (jnp.arange(npad, dtype=jnp.int32) % 1024)
    src_p = jnp.concatenate([src, padv]).reshape(_NW, _CH, 128)
    dst_p = jnp.concatenate([dst, padv]).reshape(_NW, _CH, 128)
    xd_p = jnp.pad(xd.astype(f32), ((0, _NP - _N), (0, 0)))
    xd_perm = (xd_p.reshape(_NGP, _BSP, 4, 55).transpose(0, 2, 1, 3)
               .reshape(_NP, 55))
    batch_pk = (jnp.pad(batch.astype(jnp.int32), (0, _NP - _N),
                        constant_values=300)
                .reshape(_RP, 4).transpose(1, 0)
                .reshape(4, _NGP, _BSP).transpose(1, 0, 2))

    gin = params["gin"]
    row = lambda v: v.reshape(1, -1).astype(f32)
    eye4 = jnp.eye(4, dtype=f32)
    tile4 = lambda v: jnp.tile(v.astype(f32).reshape(1, -1), (1, 4))
    bd = lambda m: jnp.kron(eye4, m.astype(f32))

    u = _mm_u0(xd_perm, gin[0]["w1"].astype(f32))
    pooled = None
    for l in range(5):
        aggp = _sc_agg(u.reshape(_NP, 32), src_p, dst_p)
        aggp = aggp.reshape(2, _RP, 128)
        p = gin[l]
        if l < 4:
            u = _mid(u, aggp, tile4(p["b1"]), bd(p["w2"]),
                     tile4(p["b2"]), tile4(p["bn_g"]), tile4(p["bn_b"]),
                     bd(gin[l + 1]["w1"]))
        else:
            pooled = _last_pool(u, aggp, tile4(p["b1"]), bd(p["w2"]),
                                tile4(p["b2"]), tile4(p["bn_g"]),
                                tile4(p["bn_b"]), batch_pk)

    xt_i = xt.astype(jnp.int32)
    wr4 = params["conv_w"].astype(f32).transpose(1, 0, 2).reshape(1000, 256)
    m3 = _prot_a(xt_i, wr4)
    mbig = (m3.reshape(26, _B, 32, 8).transpose(2, 1, 3, 0)
            .reshape(32, _B, 208))
    xt_o = _prot_b(mbig, params["emb"].astype(f32),
                   params["fc1_xt_w"].astype(f32).reshape(32, 121, 128),
                   params["conv_b"].astype(f32).reshape(32, 1, 1),
                   row(params["fc1_xt_b"]))

    w3p = jnp.pad(params["cls_w3"].astype(f32), ((0, 0), (0, 127)))
    b3p = jnp.pad(params["cls_b3"].astype(f32).reshape(1, 1),
                  ((0, 0), (0, 127)))
    head = _head(pooled, params["fc1_xd_w"].astype(f32),
                 row(params["fc1_xd_b"]), xt_o,
                 params["cls_w1"].astype(f32), row(params["cls_b1"]),
                 params["cls_w2"].astype(f32), row(params["cls_b2"]),
                 w3p, b3p)
    out = head[:, 0]
    return (out, y)


# speedup vs baseline: 2.3734x; 1.0601x over previous
import functools

import jax
import jax.numpy as jnp
from jax import lax
from jax.experimental import pallas as pl
from jax.experimental.pallas import tpu as pltpu
from jax.experimental.pallas import tpu_sc as plsc

_N = 50000
_E = 800000
_B = 256
_NP = 51200
_RPT = _NP // 16
_NW = 32
_CH = 196
_IB = 14
_NB = _CH // _IB
_EWP = _CH * 128
_BS = 512
_NG = _NP // _BS
_RP = _NP // 4
_BSP = 512
_NGP = _RP // _BSP


def _sc_agg(u, src_p, dst_p):
    mesh = plsc.VectorSubcoreMesh(
        core_axis_name="c", subcore_axis_name="s", num_cores=2, num_subcores=16
    )

    @functools.partial(
        pl.kernel,
        mesh=mesh,
        compiler_params=pltpu.CompilerParams(use_tc_tiling_on_sc=False),
        out_type=jax.ShapeDtypeStruct((2, _NP, 32), jnp.float32),
        scratch_types=[
            pltpu.VMEM((2, _IB, 128), jnp.int32),
            pltpu.VMEM((2, _IB, 128), jnp.int32),
            pltpu.VMEM((5, 128, 32), jnp.float32),
            pltpu.VMEM_SHARED((_NP, 32), jnp.float32),
            pltpu.SemaphoreType.DMA,
            pltpu.SemaphoreType.DMA,
            [pltpu.SemaphoreType.DMA] * 5,
            [pltpu.SemaphoreType.DMA] * 5,
        ],
    )
    def k(u_hbm, src_hbm, dst_hbm, out_hbm, sidx, didx, rows, agg,
          isem, zsem, gsem, ssem):
        c = lax.axis_index("c")
        s = lax.axis_index("s")
        w = s * 2 + c
        pltpu.async_copy(src_hbm.at[w, pl.ds(0, _IB)], sidx.at[0], isem)
        pltpu.async_copy(dst_hbm.at[w, pl.ds(0, _IB)], didx.at[0], isem)
        z16 = jnp.zeros((16,), jnp.float32)

        def memset(r, carry):
            rows[0, r, pl.ds(0, 16)] = z16
            rows[0, r, pl.ds(16, 16)] = z16
            return carry

        lax.fori_loop(0, 128, memset, 0)

        def zero(t, carry):
            pltpu.async_copy(rows.at[0, pl.ds(0, 128)],
                             agg.at[pl.ds(s * _RPT + t * 128, 128)], zsem)
            return carry

        lax.fori_loop(0, _RPT // 128, zero, 0)

        def zwait(t, carry):
            pltpu.make_async_copy(rows.at[0, pl.ds(0, 128)],
                                  agg.at[pl.ds(0, 128)], zsem).wait()
            return carry

        lax.fori_loop(0, _RPT // 128, zwait, 0)
        plsc.subcore_barrier()

        def gather(jp, i2, b):
            pltpu.async_copy(u_hbm.at[sidx.at[jp, i2]],
                             rows.at[b], gsem[b])

        def wait_gather(b):
            pltpu.make_async_copy(u_hbm.at[pl.ds(0, 128)], rows.at[b],
                                  gsem[b]).wait()

        def scatter(jp, i2, b):
            pltpu.async_copy(rows.at[b], agg.at[didx.at[jp, i2]],
                             ssem[b], add=True)

        def wait_scatter(b):
            pltpu.make_async_copy(rows.at[b], agg.at[pl.ds(0, 128)],
                                  ssem[b]).wait()

        def outer(j, carry):
            jp = lax.rem(j, 2)
            pltpu.make_async_copy(src_hbm.at[w, pl.ds(0, _IB)], sidx.at[jp],
                                  isem).wait()
            pltpu.make_async_copy(dst_hbm.at[w, pl.ds(0, _IB)], didx.at[jp],
                                  isem).wait()

            @pl.when(j + 1 < _NB)
            def _():
                jn = lax.rem(j + 1, 2)
                off = (j + 1) * _IB
                pltpu.async_copy(src_hbm.at[w, pl.ds(off, _IB)], sidx.at[jn],
                                 isem)
                pltpu.async_copy(dst_hbm.at[w, pl.ds(off, _IB)], didx.at[jn],
                                 isem)

            for i2 in range(4):
                gather(jp, i2, i2)
            for i2 in range(_IB):
                b = i2 % 5
                wait_gather(b)
                scatter(jp, i2, b)
                if i2 >= 1:
                    wait_scatter((i2 - 1) % 5)
                if i2 + 4 < _IB:
                    gather(jp, i2 + 4, (i2 + 4) % 5)
            wait_scatter((_IB - 1) % 5)
            return carry

        lax.fori_loop(0, _NB, outer, 0)
        plsc.subcore_barrier()
        pltpu.sync_copy(
            agg.at[pl.ds(s * _RPT, _RPT)], out_hbm.at[c, pl.ds(s * _RPT, _RPT)]
        )

    return k(u, src_p, dst_p)


def _mm_u0(xd_perm, w1):
    def body(x_ref, w_ref, o_ref):
        w = w_ref[...]
        cols = [jnp.dot(x_ref[b * _BSP:(b + 1) * _BSP], w,
                        preferred_element_type=jnp.float32)
                for b in range(4)]
        o_ref[...] = jnp.concatenate(cols, axis=1)

    return pl.pallas_call(
        body,
        grid=(_NGP,),
        in_specs=[pl.BlockSpec((4 * _BSP, 55), lambda i: (i, 0)),
                  pl.BlockSpec((55, 32), lambda i: (0, 0))],
        out_specs=pl.BlockSpec((_BSP, 128), lambda i: (i, 0)),
        out_shape=jax.ShapeDtypeStruct((_RP, 128), jnp.float32),
    )(xd_perm, w1)


def _pk_spec():
    return pl.BlockSpec((_BSP, 128), lambda i: (i, 0))


def _gin_z(u_ref, a_refs, b1_ref, w2_ref, b2_ref, g_ref, bb_ref):
    t = u_ref[...]
    for a_ref in a_refs:
        t = t + a_ref[0]
    t = jnp.maximum(t + b1_ref[...], 0.0)
    z = jnp.maximum(jnp.dot(t, w2_ref[...],
                            preferred_element_type=jnp.float32)
                    + b2_ref[...], 0.0)
    return z * g_ref[...] + bb_ref[...]


def _mid(u, aggp, b1, w2bd, b2, g, bb, w1bd):
    def body(u_ref, a0_ref, a1_ref, b1_ref, w2_ref, b2_ref, g_ref, bb_ref,
             w1n_ref, o_ref):
        i = pl.program_id(0)
        z = _gin_z(u_ref, [a0_ref, a1_ref], b1_ref, w2_ref,
                   b2_ref, g_ref, bb_ref)
        un = jnp.dot(z, w1n_ref[...], preferred_element_type=jnp.float32)
        node = (4 * (i * _BSP
                     + lax.broadcasted_iota(jnp.int32, (_BSP, 128), 0))
                + lax.broadcasted_iota(jnp.int32, (_BSP, 128), 1) // 32)
        o_ref[...] = jnp.where(node < _N, un, 0.0)

    vec = lambda: pl.BlockSpec((1, 128), lambda i: (0, 0))
    mat = lambda: pl.BlockSpec((128, 128), lambda i: (0, 0))
    agg_spec = lambda c: pl.BlockSpec((1, _BSP, 128), lambda i: (c, i, 0))
    return pl.pallas_call(
        body,
        grid=(_NGP,),
        in_specs=[_pk_spec(), agg_spec(0), agg_spec(1),
                  vec(), mat(), vec(), vec(), vec(), mat()],
        out_specs=_pk_spec(),
        out_shape=jax.ShapeDtypeStruct((_RP, 128), jnp.float32),
    )(u, aggp, aggp, b1, w2bd, b2, g, bb, w1bd)


def _last_pool(u, aggp, b1, w2bd, b2, g, bb, batch_pk):
    def body(u_ref, a0_ref, a1_ref, b1_ref, w2_ref, b2_ref, g_ref, bb_ref,
             bt_ref, o_ref):
        i = pl.program_id(0)
        z = _gin_z(u_ref, [a0_ref, a1_ref], b1_ref, w2_ref,
                   b2_ref, g_ref, bb_ref)
        bt = bt_ref[0]
        part = jnp.zeros((_B, 32), jnp.float32)
        for a in range(4):
            ids = bt[a:a + 1, :]
            oh = (ids == lax.broadcasted_iota(jnp.int32, (_B, _BSP), 0))
            part = part + jnp.dot(oh.astype(jnp.float32),
                                  z[:, 32 * a:32 * a + 32],
                                  preferred_element_type=jnp.float32)

        @pl.when(i == 0)
        def _():
            o_ref[...] = part

        @pl.when(i > 0)
        def _():
            o_ref[...] += part

    vec = lambda: pl.BlockSpec((1, 128), lambda i: (0, 0))
    mat = lambda: pl.BlockSpec((128, 128), lambda i: (0, 0))
    agg_spec = lambda c: pl.BlockSpec((1, _BSP, 128), lambda i: (c, i, 0))
    return pl.pallas_call(
        body,
        grid=(_NGP,),
        in_specs=[_pk_spec(), agg_spec(0), agg_spec(1),
                  vec(), mat(), vec(), vec(), vec(),
                  pl.BlockSpec((1, 4, _BSP), lambda i: (i, 0, 0))],
        out_specs=pl.BlockSpec((_B, 32), lambda i: (0, 0)),
        out_shape=jax.ShapeDtypeStruct((_B, 32), jnp.float32),
    )(u, aggp, aggp, b1, w2bd, b2, g, bb, batch_pk)


def _prot_a(xt, wr4):
    def body(xt_ref, w_ref, o_ref):
        c = pl.program_id(0)
        oh = (xt_ref[...] == c).astype(jnp.float32)
        o_ref[0] = jnp.dot(oh, w_ref[...], preferred_element_type=jnp.float32)

    return pl.pallas_call(
        body,
        grid=(26,),
        in_specs=[pl.BlockSpec((_B, 1000), lambda c: (0, 0)),
                  pl.BlockSpec((1000, 256), lambda c: (0, 0))],
        out_specs=pl.BlockSpec((1, _B, 256), lambda c: (c, 0, 0)),
        out_shape=jax.ShapeDtypeStruct((26, _B, 256), jnp.float32),
    )(xt, wr4)


def _prot_b(mbig, emb, fxt_w, cb, fxt_b):
    def body(m_ref, e_ref, fw_ref, cb_ref, fb_ref, o_ref):
        i = pl.program_id(0)
        e = e_ref[...]
        ekc = jnp.concatenate([e[:, k:k + 121] for k in range(8)], axis=0)
        conv = jnp.dot(m_ref[0], ekc,
                       preferred_element_type=jnp.float32) + cb_ref[0, 0, 0]
        part = jnp.dot(conv, fw_ref[0], preferred_element_type=jnp.float32)

        @pl.when(i == 0)
        def _():
            o_ref[...] = part + fb_ref[...]

        @pl.when(i > 0)
        def _():
            o_ref[...] += part

    return pl.pallas_call(
        body,
        grid=(32,),
        in_specs=[pl.BlockSpec((1, _B, 208), lambda o: (o, 0, 0)),
                  pl.BlockSpec((26, 128), lambda o: (0, 0)),
                  pl.BlockSpec((1, 121, 128), lambda o: (o, 0, 0)),
                  pl.BlockSpec((1, 1, 1), lambda o: (o, 0, 0)),
                  pl.BlockSpec((1, 128), lambda o: (0, 0))],
        out_specs=pl.BlockSpec((_B, 128), lambda o: (0, 0)),
        out_shape=jax.ShapeDtypeStruct((_B, 128), jnp.float32),
    )(mbig, emb, fxt_w, cb, fxt_b)


def _head(pooled, fxd_w, fxd_b, xt_o, w1, b1, w2, b2, w3p, b3):
    def body(p_ref, fw_ref, fb_ref, xt_ref, w1_ref, b1_ref, w2_ref, b2_ref,
             w3_ref, b3_ref, o_ref):
        xd_o = jnp.maximum(
            jnp.dot(p_ref[...], fw_ref[...],
                    preferred_element_type=jnp.float32) + fb_ref[...], 0.0)
        xj = jnp.concatenate([xd_o, xt_ref[...]], axis=1)
        h2 = jnp.maximum(jnp.dot(xj, w1_ref[...],
                                 preferred_element_type=jnp.float32)
                         + b1_ref[...], 0.0)
        h2 = jnp.maximum(jnp.dot(h2, w2_ref[...],
                                 preferred_element_type=jnp.float32)
                         + b2_ref[...], 0.0)
        o_ref[...] = jnp.dot(h2, w3_ref[...],
                             preferred_element_type=jnp.float32) + b3_ref[...]

    full = lambda shp: pl.BlockSpec(shp, lambda: tuple(0 for _ in shp))
    return pl.pallas_call(
        body,
        in_specs=[full((_B, 32)), full((32, 128)), full((1, 128)),
                  full((_B, 128)), full((256, 1024)), full((1, 1024)),
                  full((1024, 256)), full((1, 256)), full((256, 128)),
                  full((1, 128))],
        out_specs=full((_B, 128)),
        out_shape=jax.ShapeDtypeStruct((_B, 128), jnp.float32),
    )(pooled, fxd_w, fxd_b, xt_o, w1, b1, w2, b2, w3p, b3)


def kernel(xd, edge_index, batch, xt, y, params):
    f32 = jnp.float32
    src = edge_index[0].astype(jnp.int32)
    dst = edge_index[1].astype(jnp.int32)
    npad = _NW * _EWP - _E
    padv = _N + (jnp.arange(npad, dtype=jnp.int32) % 1024)
    src_p = jnp.concatenate([src, padv]).reshape(_NW, _CH, 128)
    dst_p = jnp.concatenate([dst, padv]).reshape(_NW, _CH, 128)
    xd_p = jnp.pad(xd.astype(f32), ((0, _NP - _N), (0, 0)))
    xd_perm = (xd_p.reshape(_NGP, _BSP, 4, 55).transpose(0, 2, 1, 3)
               .reshape(_NP, 55))
    batch_pk = (jnp.pad(batch.astype(jnp.int32), (0, _NP - _N),
                        constant_values=300)
                .reshape(_RP, 4).transpose(1, 0)
                .reshape(4, _NGP, _BSP).transpose(1, 0, 2))

    gin = params["gin"]
    row = lambda v: v.reshape(1, -1).astype(f32)
    eye4 = jnp.eye(4, dtype=f32)
    tile4 = lambda v: jnp.tile(v.astype(f32).reshape(1, -1), (1, 4))
    bd = lambda m: jnp.kron(eye4, m.astype(f32))

    u = _mm_u0(xd_perm, gin[0]["w1"].astype(f32))
    pooled = None
    for l in range(5):
        aggp = _sc_agg(u.reshape(_NP, 32), src_p, dst_p)
        aggp = aggp.reshape(2, _RP, 128)
        p = gin[l]
        if l < 4:
            u = _mid(u, aggp, tile4(p["b1"]), bd(p["w2"]),
                     tile4(p["b2"]), tile4(p["bn_g"]), tile4(p["bn_b"]),
                     bd(gin[l + 1]["w1"]))
        else:
            pooled = _last_pool(u, aggp, tile4(p["b1"]), bd(p["w2"]),
                                tile4(p["b2"]), tile4(p["bn_g"]),
                                tile4(p["bn_b"]), batch_pk)

    xt_i = xt.astype(jnp.int32)
    wr4 = params["conv_w"].astype(f32).transpose(1, 0, 2).reshape(1000, 256)
    m3 = _prot_a(xt_i, wr4)
    mbig = (m3.reshape(26, _B, 32, 8).transpose(2, 1, 3, 0)
            .reshape(32, _B, 208))
    xt_o = _prot_b(mbig, params["emb"].astype(f32),
                   params["fc1_xt_w"].astype(f32).reshape(32, 121, 128),
                   params["conv_b"].astype(f32).reshape(32, 1, 1),
                   row(params["fc1_xt_b"]))

    w3p = jnp.pad(params["cls_w3"].astype(f32), ((0, 0), (0, 127)))
    b3p = jnp.pad(params["cls_b3"].astype(f32).reshape(1, 1),
                  ((0, 0), (0, 127)))
    head = _head(pooled, params["fc1_xd_w"].astype(f32),
                 row(params["fc1_xd_b"]), xt_o,
                 params["cls_w1"].astype(f32), row(params["cls_b1"]),
                 params["cls_w2"].astype(f32), row(params["cls_b2"]),
                 w3p, b3p)
    out = head[:, 0]
    return (out, y)
